# Initial kernel scaffold; baseline (speedup 1.0000x reference)
import jax
import jax.numpy as jnp
from jax.experimental import pallas as pl


def kernel(x, gate_W1, gate_b1, gate_W2, gate_b2, W1, b1, W2, b2, W3, b3):
    raise NotImplementedError("write your pallas kernel here")


# speedup vs baseline: 1.6432x; 1.6432x over previous
import jax
import jax.numpy as jnp
from jax.experimental import pallas as pl
from jax.experimental.pallas import tpu as pltpu

_N = 2048
_D = 768
_H = 128
_GH = 64
_E = 16
_BALANCE_COEF = 0.01
_NEG = -1e30


def _moe_body(x_ref, gw1_ref, gb1_ref, gw2_ref, gb2_ref,
              w1_ref, b1_ref, w2_ref, b2_ref, w3_ref, b3_ref,
              out_ref, usage_ref, loss_ref, combine_ref):
    e = pl.program_id(0)

    @pl.when(e == 0)
    def _gating():
        x = x_ref[...]
        gh = jnp.maximum(
            jnp.dot(x, gw1_ref[...], preferred_element_type=jnp.float32)
            + gb1_ref[...], 0.0)
        logits = (jnp.dot(gh, gw2_ref[...], preferred_element_type=jnp.float32)
                  + gb2_ref[...])
        m = jnp.max(logits, axis=1, keepdims=True)
        p = jnp.exp(logits - m)
        p = p / jnp.sum(p, axis=1, keepdims=True)
        lane = jax.lax.broadcasted_iota(jnp.int32, (_N, _E), 1)
        m0 = jnp.max(p, axis=1, keepdims=True)
        idx0 = jnp.min(jnp.where(p == m0, lane, _E), axis=1, keepdims=True)
        mask0 = lane == idx0
        pm = jnp.where(mask0, _NEG, p)
        m1 = jnp.max(pm, axis=1, keepdims=True)
        idx1 = jnp.min(jnp.where(pm == m1, lane, _E), axis=1, keepdims=True)
        mask1 = lane == idx1
        denom = m0 + m1
        combine_ref[...] = (jnp.where(mask0, m0, 0.0)
                            + jnp.where(mask1, m1, 0.0)) / denom
        sel = mask0.astype(jnp.float32) + mask1.astype(jnp.float32)
        usage = jnp.sum(sel, axis=0) / _N
        usage_ref[...] = usage.reshape(1, _E)
        loss_ref[...] = (jnp.mean((usage - 1.0 / _E) ** 2)
                         * _BALANCE_COEF).reshape(1, 1)
        out_ref[...] = jnp.zeros_like(out_ref)

    x = x_ref[...]
    h1 = jnp.maximum(
        jnp.dot(x, w1_ref[0], preferred_element_type=jnp.float32)
        + b1_ref[0], 0.0)
    h2 = jnp.maximum(
        jnp.dot(h1, w2_ref[0], preferred_element_type=jnp.float32)
        + b2_ref[0], 0.0)
    ye = (jnp.dot(h2, w3_ref[0], preferred_element_type=jnp.float32)
          + b3_ref[0])
    lane = jax.lax.broadcasted_iota(jnp.int32, (_N, _E), 1)
    coeff = jnp.sum(jnp.where(lane == e, combine_ref[...], 0.0),
                    axis=1, keepdims=True)
    out_ref[...] += coeff * ye


def kernel(x, gate_W1, gate_b1, gate_W2, gate_b2, W1, b1, W2, b2, W3, b3):
    out, usage, loss = pl.pallas_call(
        _moe_body,
        grid=(_E,),
        in_specs=[
            pl.BlockSpec((_N, _D), lambda e: (0, 0)),
            pl.BlockSpec((_D, _GH), lambda e: (0, 0)),
            pl.BlockSpec((1, _GH), lambda e: (0, 0)),
            pl.BlockSpec((_GH, _E), lambda e: (0, 0)),
            pl.BlockSpec((1, _E), lambda e: (0, 0)),
            pl.BlockSpec((1, _D, _H), lambda e: (e, 0, 0)),
            pl.BlockSpec((1, 1, _H), lambda e: (e, 0, 0)),
            pl.BlockSpec((1, _H, _H), lambda e: (e, 0, 0)),
            pl.BlockSpec((1, 1, _H), lambda e: (e, 0, 0)),
            pl.BlockSpec((1, _H, _D), lambda e: (e, 0, 0)),
            pl.BlockSpec((1, 1, _D), lambda e: (e, 0, 0)),
        ],
        out_specs=[
            pl.BlockSpec((_N, _D), lambda e: (0, 0)),
            pl.BlockSpec((1, _E), lambda e: (0, 0)),
            pl.BlockSpec((1, 1), lambda e: (0, 0)),
        ],
        out_shape=[
            jax.ShapeDtypeStruct((_N, _D), jnp.float32),
            jax.ShapeDtypeStruct((1, _E), jnp.float32),
            jax.ShapeDtypeStruct((1, 1), jnp.float32),
        ],
        scratch_shapes=[pltpu.VMEM((_N, _E), jnp.float32)],
    )(x, gate_W1, gate_b1.reshape(1, _GH), gate_W2, gate_b2.reshape(1, _E),
      W1, b1.reshape(_E, 1, _H), W2, b2.reshape(_E, 1, _H),
      W3, b3.reshape(_E, 1, _D))
    return out, loss.reshape(()), usage.reshape(_E)
